# baseline (speedup 1.0000x reference)
import jax
import jax.numpy as jnp
from jax import lax
from jax.experimental import pallas as pl
from jax.experimental.pallas import tpu as pltpu
from jax.experimental.pallas import tpu_sc as plsc

N = 10000
E = 320000
DF = 128
H = 16
OUT = 7

NC = 2
NS = 16
NW = NC * NS
CH = 64
NSHARD = 16
EPS = E // NSHARD
NCHUNK = 320
NSTAGE = 4
STG = NCHUNK // NSTAGE
SPAD = NCHUNK * CH - EPS

HALFN = 5120
WIN = HALFN
ACC2 = NC * WIN
RPT = WIN // NS
NZP = RPT // CH
TAB_ROWS = N + 240
ZROW = N


def _mm1_body(f_ref, w_ref, o_ref):
    g = jnp.dot(f_ref[...], w_ref[...], preferred_element_type=jnp.float32)
    o_ref[...] = jnp.zeros((TAB_ROWS, DF), jnp.float32)
    o_ref[:N, :H] = g


def _mid_body(p_ref, b1_ref, w2_ref, o_ref):
    h1 = jnp.concatenate([p_ref[0, :HALFN, :H],
                          p_ref[1, : N - HALFN, :H]], axis=0)
    h = jnp.maximum(h1 + b1_ref[...], 0.0)
    g = jnp.dot(h, w2_ref[...], preferred_element_type=jnp.float32)
    o_ref[...] = jnp.zeros((TAB_ROWS, DF), jnp.float32)
    o_ref[:N, :H] = g


def _fin_body(q_ref, b2_ref, o_ref):
    h2 = jnp.concatenate([q_ref[0, :HALFN, :OUT],
                          q_ref[1, : N - HALFN, :OUT]], axis=0)
    o_ref[...] = h2 + b2_ref[...]


def _copy_idx_row(src2d, j, dst1d):
    for k in range(CH // 16):
        dst1d[pl.ds(k * 16, 16)] = src2d[j, pl.ds(k * 16, 16)]


def _agg_body(g_hbm, src_hbm, dst_hbm, ids_hbm, out_hbm,
              src_v, dst_v, ids_v, idx_cur, rows_a, rows_b,
              acc_sh, sem_a, sem_b):
    cid = lax.axis_index("c")
    sid = lax.axis_index("s")
    w = cid * NS + sid

    pltpu.sync_copy(ids_hbm.at[w], ids_v)

    def _z(i, carry):
        def _zz(k, c2):
            rows_a[i, pl.ds(k * 16, 16)] = jnp.zeros((16,), jnp.float32)
            return c2
        lax.fori_loop(0, DF // 16, _zz, 0)
        return carry

    lax.fori_loop(0, CH, _z, 0)
    for t in range(NZP):
        _copy_idx_row(ids_v, t, idx_cur)
        pltpu.sync_copy(rows_a, acc_sh.at[idx_cur])
    plsc.subcore_barrier()

    def _step(j, carry):
        even = lax.rem(j, 2) == 0
        _copy_idx_row(dst_v, j, idx_cur)

        @pl.when(even)
        def _():
            pltpu.make_async_copy(g_hbm.at[src_v.at[j]], rows_a, sem_a).wait()

            @pl.when(j + 1 < STG)
            def _():
                pltpu.async_copy(g_hbm.at[src_v.at[j + 1]], rows_b, sem_b)
            pltpu.sync_copy(rows_a, acc_sh.at[idx_cur], add=True)

        @pl.when(jnp.logical_not(even))
        def _():
            pltpu.make_async_copy(g_hbm.at[src_v.at[j]], rows_b, sem_b).wait()

            @pl.when(j + 1 < STG)
            def _():
                pltpu.async_copy(g_hbm.at[src_v.at[j + 1]], rows_a, sem_a)
            pltpu.sync_copy(rows_b, acc_sh.at[idx_cur], add=True)

        return carry

    for h in range(NSTAGE):
        pltpu.sync_copy(src_hbm.at[cid, sid, pl.ds(h * STG, STG)], src_v)
        pltpu.sync_copy(dst_hbm.at[cid, sid, pl.ds(h * STG, STG)], dst_v)
        pltpu.async_copy(g_hbm.at[src_v.at[0]], rows_a, sem_a)
        lax.fori_loop(0, STG, _step, 0)
    plsc.subcore_barrier()

    r0 = sid * RPT
    for t in range(NZP):
        pltpu.async_copy(acc_sh.at[ids_v.at[t]], rows_a, sem_a).wait()
        pltpu.sync_copy(rows_a, out_hbm.at[cid, pl.ds(r0 + t * CH, CH)])


def _make_agg():
    mesh = plsc.VectorSubcoreMesh(core_axis_name="c", subcore_axis_name="s")
    return pl.kernel(
        _agg_body,
        out_type=jax.ShapeDtypeStruct((NC, WIN, DF), jnp.float32),
        mesh=mesh,
        scratch_types=[
            pltpu.VMEM((STG, CH), jnp.int32),
            pltpu.VMEM((STG, CH), jnp.int32),
            pltpu.VMEM((NZP, CH), jnp.int32),
            pltpu.VMEM((CH,), jnp.int32),
            pltpu.VMEM((CH, DF), jnp.float32),
            pltpu.VMEM((CH, DF), jnp.float32),
            pltpu.VMEM_SHARED((ACC2, DF), jnp.float32),
            pltpu.SemaphoreType.DMA,
            pltpu.SemaphoreType.DMA,
        ],
    )


@jax.jit
def kernel(features, edge_index, W1, b1, W2, b2):
    src = edge_index[0]
    dst = edge_index[1]

    srcsh = src.reshape(NSHARD, EPS)
    dstsh = dst.reshape(NSHARD, EPS)
    in0 = dstsh < HALFN
    s0 = jnp.where(in0, srcsh, ZROW)
    d0 = jnp.where(in0, dstsh, 0)
    in1 = dstsh >= HALFN
    s1 = jnp.where(in1, srcsh, ZROW)
    d1 = jnp.where(in1, dstsh, HALFN)
    spad = jnp.full((NSHARD, SPAD), ZROW, jnp.int32)
    src4 = jnp.stack([
        jnp.concatenate([s0, spad], axis=1).reshape(NSHARD, NCHUNK, CH),
        jnp.concatenate([s1, spad], axis=1).reshape(NSHARD, NCHUNK, CH),
    ]).astype(jnp.int32)
    dst4 = jnp.stack([
        jnp.concatenate([d0, jnp.zeros((NSHARD, SPAD), jnp.int32)],
                        axis=1).reshape(NSHARD, NCHUNK, CH),
        jnp.concatenate([d1, jnp.full((NSHARD, SPAD), HALFN, jnp.int32)],
                        axis=1).reshape(NSHARD, NCHUNK, CH),
    ]).astype(jnp.int32)

    ids3 = (jnp.arange(NW, dtype=jnp.int32)[:, None, None] * RPT
            + jnp.arange(NZP, dtype=jnp.int32)[None, :, None] * CH
            + jnp.arange(CH, dtype=jnp.int32)[None, None, :])

    g1 = pl.pallas_call(
        _mm1_body,
        out_shape=jax.ShapeDtypeStruct((TAB_ROWS, DF), jnp.float32),
    )(features, W1)

    agg = _make_agg()
    p = agg(g1, src4, dst4, ids3)

    w2p = jnp.zeros((H, H), jnp.float32).at[:, :OUT].set(W2)
    g2 = pl.pallas_call(
        _mid_body,
        out_shape=jax.ShapeDtypeStruct((TAB_ROWS, DF), jnp.float32),
    )(p, b1.reshape(1, H), w2p)

    q = agg(g2, src4, dst4, ids3)

    out = pl.pallas_call(
        _fin_body,
        out_shape=jax.ShapeDtypeStruct((N, OUT), jnp.float32),
    )(q, b2.reshape(1, OUT))
    return out


# speedup vs baseline: 7.8288x; 7.8288x over previous
import jax
import jax.numpy as jnp
from jax import lax
from jax.experimental import pallas as pl
from jax.experimental.pallas import tpu as pltpu
from jax.experimental.pallas import tpu_sc as plsc

N = 10000
E = 320000
DF = 128
H = 16
OUT = 7

NC = 2
NS = 16
NW = NC * NS
CH = 128
NSHARD = 16
EPS = E // NSHARD
NCHUNK = 160
SPAD = NCHUNK * CH - EPS

HALFN = 5120
WIN = HALFN
ACC2 = NC * WIN
RPT = WIN // NS
NZP = 3
TAB_ROWS = N + 240
ZROW = N


def _mm1_body(f_ref, w_ref, o_ref):
    g = jnp.dot(f_ref[...], w_ref[...], preferred_element_type=jnp.float32)
    o_ref[...] = jnp.zeros((TAB_ROWS, H), jnp.float32)
    o_ref[:N, :] = g


def _mid_body(p_ref, b1_ref, w2_ref, o_ref):
    h1 = jnp.concatenate([p_ref[0], p_ref[1, : N - HALFN]], axis=0)
    h = jnp.maximum(h1 + b1_ref[...], 0.0)
    g = jnp.dot(h, w2_ref[...], preferred_element_type=jnp.float32)
    o_ref[...] = jnp.zeros((TAB_ROWS, H), jnp.float32)
    o_ref[:N, :] = g


def _fin_body(q_ref, b2_ref, o_ref):
    h2 = jnp.concatenate([q_ref[0, :, :OUT],
                          q_ref[1, : N - HALFN, :OUT]], axis=0)
    o_ref[...] = h2 + b2_ref[...]


def _copy_idx_row(src2d, j, dst1d):
    for k in range(CH // 16):
        dst1d[pl.ds(k * 16, 16)] = src2d[j, pl.ds(k * 16, 16)]


def _agg_body(g_hbm, src_hbm, dst_hbm, ids_hbm, out_hbm,
              src_v, dst_v, ids_v, idx_cur, rows_a, rows_b,
              acc_sh, sem_a, sem_b):
    cid = lax.axis_index("c")
    sid = lax.axis_index("s")
    w = cid * NS + sid

    pltpu.sync_copy(ids_hbm.at[w], ids_v)
    pltpu.sync_copy(src_hbm.at[cid, sid], src_v)
    pltpu.sync_copy(dst_hbm.at[cid, sid], dst_v)

    def _z(i, carry):
        rows_a[i] = jnp.zeros((16,), jnp.float32)
        return carry

    lax.fori_loop(0, CH, _z, 0)
    for t in range(NZP):
        _copy_idx_row(ids_v, t, idx_cur)
        pltpu.sync_copy(rows_a, acc_sh.at[idx_cur])
    plsc.subcore_barrier()

    def _step(j, carry):
        even = lax.rem(j, 2) == 0
        _copy_idx_row(dst_v, j, idx_cur)

        @pl.when(even)
        def _():
            pltpu.make_async_copy(g_hbm.at[src_v.at[j]], rows_a, sem_a).wait()

            @pl.when(j + 1 < NCHUNK)
            def _():
                pltpu.async_copy(g_hbm.at[src_v.at[j + 1]], rows_b, sem_b)
            pltpu.sync_copy(rows_a, acc_sh.at[idx_cur], add=True)

        @pl.when(jnp.logical_not(even))
        def _():
            pltpu.make_async_copy(g_hbm.at[src_v.at[j]], rows_b, sem_b).wait()

            @pl.when(j + 1 < NCHUNK)
            def _():
                pltpu.async_copy(g_hbm.at[src_v.at[j + 1]], rows_a, sem_a)
            pltpu.sync_copy(rows_b, acc_sh.at[idx_cur], add=True)

        return carry

    pltpu.async_copy(g_hbm.at[src_v.at[0]], rows_a, sem_a)
    lax.fori_loop(0, NCHUNK, _step, 0)
    plsc.subcore_barrier()

    r0 = sid * RPT
    for t in range(NZP):
        nrows = CH if t < NZP - 1 else RPT - (NZP - 1) * CH
        pltpu.async_copy(acc_sh.at[ids_v.at[t]], rows_a, sem_a).wait()
        pltpu.sync_copy(rows_a.at[pl.ds(0, nrows)],
                        out_hbm.at[cid, pl.ds(r0 + t * CH, nrows)])


def _make_agg():
    mesh = plsc.VectorSubcoreMesh(core_axis_name="c", subcore_axis_name="s")
    return pl.kernel(
        _agg_body,
        out_type=jax.ShapeDtypeStruct((NC, WIN, H), jnp.float32),
        mesh=mesh,
        scratch_types=[
            pltpu.VMEM((NCHUNK, CH), jnp.int32),
            pltpu.VMEM((NCHUNK, CH), jnp.int32),
            pltpu.VMEM((NZP, CH), jnp.int32),
            pltpu.VMEM((CH,), jnp.int32),
            pltpu.VMEM((CH, H), jnp.float32),
            pltpu.VMEM((CH, H), jnp.float32),
            pltpu.VMEM_SHARED((ACC2, H), jnp.float32),
            pltpu.SemaphoreType.DMA,
            pltpu.SemaphoreType.DMA,
        ],
        compiler_params=pltpu.CompilerParams(use_tc_tiling_on_sc=False),
    )


@jax.jit
def kernel(features, edge_index, W1, b1, W2, b2):
    src = edge_index[0]
    dst = edge_index[1]

    srcsh = src.reshape(NSHARD, EPS)
    dstsh = dst.reshape(NSHARD, EPS)
    in0 = dstsh < HALFN
    s0 = jnp.where(in0, srcsh, ZROW)
    d0 = jnp.where(in0, dstsh, 0)
    in1 = dstsh >= HALFN
    s1 = jnp.where(in1, srcsh, ZROW)
    d1 = jnp.where(in1, dstsh, HALFN)
    spad = jnp.full((NSHARD, SPAD), ZROW, jnp.int32)
    src4 = jnp.stack([
        jnp.concatenate([s0, spad], axis=1).reshape(NSHARD, NCHUNK, CH),
        jnp.concatenate([s1, spad], axis=1).reshape(NSHARD, NCHUNK, CH),
    ]).astype(jnp.int32)
    dst4 = jnp.stack([
        jnp.concatenate([d0, jnp.zeros((NSHARD, SPAD), jnp.int32)],
                        axis=1).reshape(NSHARD, NCHUNK, CH),
        jnp.concatenate([d1, jnp.full((NSHARD, SPAD), HALFN, jnp.int32)],
                        axis=1).reshape(NSHARD, NCHUNK, CH),
    ]).astype(jnp.int32)

    lane = jnp.arange(CH, dtype=jnp.int32)
    t2 = 2 * CH + (lane % (RPT - 2 * CH))
    ids3 = (jnp.arange(NW, dtype=jnp.int32)[:, None, None] * RPT
            + jnp.stack([lane, CH + lane, t2])[None, :, :])

    g1 = pl.pallas_call(
        _mm1_body,
        out_shape=jax.ShapeDtypeStruct((TAB_ROWS, H), jnp.float32),
    )(features, W1)

    agg = _make_agg()
    p = agg(g1, src4, dst4, ids3)

    w2p = jnp.zeros((H, H), jnp.float32).at[:, :OUT].set(W2)
    g2 = pl.pallas_call(
        _mid_body,
        out_shape=jax.ShapeDtypeStruct((TAB_ROWS, H), jnp.float32),
    )(p, b1.reshape(1, H), w2p)

    q = agg(g2, src4, dst4, ids3)

    out = pl.pallas_call(
        _fin_body,
        out_shape=jax.ShapeDtypeStruct((N, OUT), jnp.float32),
    )(q, b2.reshape(1, OUT))
    return out
